# baseline (speedup 1.0000x reference)
import functools

import jax
import jax.numpy as jnp
from jax.experimental import pallas as pl
from jax.experimental.pallas import tpu as pltpu

_TILE = 4096
_MATCH_EPS = 1e-4
_BIG = 2 ** 30


def _merge_tile(q2, kt, lbl, gbase, k_total, minval_ref, minpk_ref):
    tile = kt.shape[0]
    ones8 = jnp.ones((8, kt.shape[1]), jnp.float32)
    k_sq8 = jnp.dot(ones8, (kt * kt).T, preferred_element_type=jnp.float32)
    col = jax.lax.broadcasted_iota(jnp.int32, (1, tile), 1)
    gcol = gbase + col
    k_sq_row = jnp.where(gcol < k_total, k_sq8[0:1, :], jnp.inf)
    prod2 = jnp.dot(q2, kt.T, preferred_element_type=jnp.float32)
    s = k_sq_row + prod2
    packed_row = (gcol << 4) | lbl[None, :]

    prev = minval_ref[...]
    better = s < prev
    minval_ref[...] = jnp.minimum(s, prev)
    minpk_ref[...] = jnp.where(better, packed_row, minpk_ref[...])


def _knn_body(q_ref, ka_ref, kb_ref, la_ref, lb_ref, qlbl_ref,
              pred_ref, acc_ref, minval_ref, minpk_ref,
              *, n_steps, k_total):
    i = pl.program_id(0)
    tile = _TILE

    @pl.when(i == 0)
    def _init():
        minval_ref[...] = jnp.full(minval_ref.shape, jnp.inf, jnp.float32)
        minpk_ref[...] = jnp.full(minpk_ref.shape, jnp.int32(_BIG))

    q = q_ref[...]
    q2 = q * -2.0
    _merge_tile(q2, ka_ref[...], la_ref[0, 0, :], (2 * i) * tile,
                k_total, minval_ref, minpk_ref)
    _merge_tile(q2, kb_ref[...], lb_ref[0, 0, :], (2 * i + 1) * tile,
                k_total, minval_ref, minpk_ref)

    @pl.when(i == n_steps - 1)
    def _epilogue():
        mv = minval_ref[...]
        mpk = minpk_ref[...]
        best = jnp.min(mv, axis=1, keepdims=True)
        cand = jnp.where(mv == best, mpk, jnp.int32(_BIG))
        bestpk = jnp.min(cand, axis=1, keepdims=True)
        label = bestpk & 15
        q_sq = jnp.sum(q * q, axis=1, keepdims=True)
        matched = (best + q_sq) < _MATCH_EPS
        pred = jnp.where(matched, label, jnp.int32(0))
        pred_ref[...] = pred
        correct = (pred == qlbl_ref[...]).astype(jnp.float32)
        acc_ref[0, 0] = jnp.sum(correct) / correct.shape[0]


def kernel(queries, keys, memory_labels, query_labels):
    q_n, d = queries.shape
    k_total = keys.shape[0]
    tile = _TILE
    n_steps = -(-k_total // (2 * tile))
    k_pad = 2 * n_steps * tile

    keys_p = jnp.pad(keys, ((0, k_pad - k_total), (0, 0)))
    lbl_p = jnp.pad(memory_labels, (0, k_pad - k_total))
    keys_t = keys_p.reshape(n_steps, 2, tile, d)
    ka = keys_t[:, 0].reshape(n_steps * tile, d)
    kb = keys_t[:, 1].reshape(n_steps * tile, d)
    lbl_t = lbl_p.reshape(n_steps, 2, 1, tile)
    la = lbl_t[:, 0].reshape(n_steps, 1, tile)
    lb = lbl_t[:, 1].reshape(n_steps, 1, tile)
    qlbl = query_labels.reshape(q_n, 1)

    body = functools.partial(_knn_body, n_steps=n_steps, k_total=k_total)
    pred, acc = pl.pallas_call(
        body,
        grid=(n_steps,),
        in_specs=[
            pl.BlockSpec((q_n, d), lambda i: (0, 0)),
            pl.BlockSpec((tile, d), lambda i: (i, 0)),
            pl.BlockSpec((tile, d), lambda i: (i, 0)),
            pl.BlockSpec((1, 1, tile), lambda i: (i, 0, 0)),
            pl.BlockSpec((1, 1, tile), lambda i: (i, 0, 0)),
            pl.BlockSpec((q_n, 1), lambda i: (0, 0)),
        ],
        out_specs=[
            pl.BlockSpec((q_n, 1), lambda i: (0, 0)),
            pl.BlockSpec(memory_space=pltpu.SMEM),
        ],
        out_shape=[
            jax.ShapeDtypeStruct((q_n, 1), jnp.int32),
            jax.ShapeDtypeStruct((1, 1), jnp.float32),
        ],
        scratch_shapes=[
            pltpu.VMEM((q_n, tile), jnp.float32),
            pltpu.VMEM((q_n, tile), jnp.int32),
        ],
    )(queries, ka, kb, la, lb, qlbl)

    return pred[:, 0], acc[0, 0]
